# baseline (speedup 1.0000x reference)
import functools

import jax
import jax.numpy as jnp
from jax import lax
from jax.experimental import pallas as pl
from jax.experimental.pallas import tpu as pltpu
from jax.experimental.pallas import tpu_sc as plsc

_N = 10000
_R = 16
_D = 128
_NB = 30

_NC = 2
_NS = 16
_NW = _NC * _NS
_CH = 128
_L = 16

_HIST = 160256
_HB = _HIST // _NS
_HW = _HIST // _NW
_NPAD = 10112
_RT = _NPAD // _NS

_BN = 400
_NBLK = _N // _BN


def _sc_mesh():
    return plsc.VectorSubcoreMesh(core_axis_name="c", subcore_axis_name="s")


def _zero16(ref, nwords):
    def z16(j, c2):
        ref[pl.ds(j * _L, _L)] = jnp.zeros((_L,), jnp.float32)
        return c2

    lax.fori_loop(0, nwords // _L, z16, 0)


def _make_hist(per_tile):
    nchunk = per_tile // _CH

    def body(dst_hbm, typ_hbm, out_hbm, hist_sh, histv, dstv, typv, segv,
             onesv, sem):
        c = lax.axis_index("c")
        s = lax.axis_index("s")
        wid = c * _NS + s

        _zero16(histv, _HB)
        pltpu.sync_copy(histv, hist_sh.at[pl.ds(s * _HB, _HB)])
        for j in range(_CH // _L):
            onesv[pl.ds(j * _L, _L)] = jnp.ones((_L,), jnp.float32)
        plsc.subcore_barrier()

        base = wid * per_tile

        def chunk(i, carry):
            off = base + i * _CH
            pltpu.sync_copy(dst_hbm.at[pl.ds(off, _CH)], dstv)
            pltpu.sync_copy(typ_hbm.at[pl.ds(off, _CH)], typv)

            def seg16(j, c2):
                sl = pl.ds(j * _L, _L)
                segv[sl] = dstv[sl] * _R + typv[sl]
                return c2

            lax.fori_loop(0, _CH // _L, seg16, 0)
            pltpu.sync_copy(onesv, hist_sh.at[segv], add=True)
            return carry

        lax.fori_loop(0, nchunk, chunk, 0)
        plsc.subcore_barrier()
        pltpu.sync_copy(hist_sh.at[pl.ds(s * _HB, _HB)], histv)
        pltpu.sync_copy(histv, out_hbm.at[pl.ds(c * _HIST + s * _HB, _HB)])

    return pl.kernel(
        body,
        out_type=jax.ShapeDtypeStruct((_NC * _HIST,), jnp.float32),
        mesh=_sc_mesh(),
        scratch_types=[
            pltpu.VMEM_SHARED((_HIST,), jnp.float32),
            pltpu.VMEM((_HB,), jnp.float32),
            pltpu.VMEM((_CH,), jnp.int32),
            pltpu.VMEM((_CH,), jnp.int32),
            pltpu.VMEM((_CH,), jnp.int32),
            pltpu.VMEM((_CH,), jnp.float32),
            pltpu.SemaphoreType.DMA,
        ],
    )


def _make_prep(per_tile):
    nchunk = per_tile // _CH

    def body(src_hbm, typ_hbm, dst_hbm, h0_hbm, h1_hbm, gidx_hbm, w_hbm,
             srcv, typv, dstv, segv, gidxv, p0v, p1v, wv, sem):
        c = lax.axis_index("c")
        s = lax.axis_index("s")
        wid = c * _NS + s
        base = wid * per_tile

        def chunk(i, carry):
            off = base + i * _CH
            pltpu.sync_copy(src_hbm.at[pl.ds(off, _CH)], srcv)
            pltpu.sync_copy(typ_hbm.at[pl.ds(off, _CH)], typv)
            pltpu.sync_copy(dst_hbm.at[pl.ds(off, _CH)], dstv)

            def idx16(j, c2):
                sl = pl.ds(j * _L, _L)
                gidxv[sl] = typv[sl] * _N + srcv[sl]
                segv[sl] = dstv[sl] * _R + typv[sl]
                return c2

            lax.fori_loop(0, _CH // _L, idx16, 0)
            pltpu.sync_copy(gidxv, gidx_hbm.at[pl.ds(off, _CH)])
            pltpu.async_copy(h0_hbm.at[segv], p0v, sem).wait()
            pltpu.async_copy(h1_hbm.at[segv], p1v, sem).wait()

            def w16(j, c2):
                sl = pl.ds(j * _L, _L)
                wv[sl] = 1.0 / jnp.maximum(p0v[sl] + p1v[sl], 1.0)
                return c2

            lax.fori_loop(0, _CH // _L, w16, 0)
            pltpu.sync_copy(wv, w_hbm.at[pl.ds(off, _CH)])
            return carry

        lax.fori_loop(0, nchunk, chunk, 0)

    return pl.kernel(
        body,
        out_type=(jax.ShapeDtypeStruct((per_tile * _NW,), jnp.int32),
                  jax.ShapeDtypeStruct((per_tile * _NW,), jnp.float32)),
        mesh=_sc_mesh(),
        scratch_types=[
            pltpu.VMEM((_CH,), jnp.int32),
            pltpu.VMEM((_CH,), jnp.int32),
            pltpu.VMEM((_CH,), jnp.int32),
            pltpu.VMEM((_CH,), jnp.int32),
            pltpu.VMEM((_CH,), jnp.int32),
            pltpu.VMEM((_CH,), jnp.float32),
            pltpu.VMEM((_CH,), jnp.float32),
            pltpu.VMEM((_CH,), jnp.float32),
            pltpu.SemaphoreType.DMA,
        ],
    )


_G = 16


def _make_scatter(per_tile, n0):
    nchunk = per_tile // _CH
    n1 = 2 * nchunk - n0

    def body(h_hbm, gidx_hbm, w_hbm, dst_hbm, out_hbm,
             agg_sh, zb, gidxv, dstv, wv_, rows0, sem):
        c = lax.axis_index("c")
        s = lax.axis_index("s")

        for k in range(8):
            for j in range(_D // _L):
                zb[k, pl.ds(j * _L, _L)] = jnp.zeros((_L,), jnp.float32)

        def zrow(i, c2):
            pltpu.sync_copy(zb, agg_sh.at[pl.ds(s * _RT + i * 8, 8)])
            return c2

        lax.fori_loop(0, _RT // 8, zrow, 0)
        plsc.subcore_barrier()

        base = jnp.where(c == 0, s * n0, _NS * n0 + s * n1) * _CH
        nch = jnp.where(c == 0, n0, n1)

        def chunk(i, carry):
            off = base + i * _CH
            pltpu.sync_copy(gidx_hbm.at[pl.ds(off, _CH)], gidxv)
            pltpu.sync_copy(w_hbm.at[pl.ds(off, _CH)], wv_)
            pltpu.sync_copy(dst_hbm.at[pl.ds(off, _CH)], dstv)
            pltpu.async_copy(h_hbm.at[gidxv], rows0, sem).wait()

            def sgrp(gg, c2):
                wv = wv_[pl.ds(gg * _L, _L)]
                for k in range(_L):
                    wsc = wv[k]
                    rr = gg * _L + k
                    for j in range(_D // _L):
                        sl = pl.ds(j * _L, _L)
                        rows0[rr, sl] = rows0[rr, sl] * wsc
                return c2

            lax.fori_loop(0, _CH // _L, sgrp, 0)
            pltpu.sync_copy(rows0, agg_sh.at[dstv], add=True)
            return carry

        lax.fori_loop(0, nch, chunk, 0)
        plsc.subcore_barrier()
        o = 0
        while o < _RT:
            sz = min(_CH, _RT - o)
            pltpu.sync_copy(agg_sh.at[pl.ds(s * _RT + o, sz)],
                            rows0.at[pl.ds(0, sz)])
            pltpu.sync_copy(rows0.at[pl.ds(0, sz)],
                            out_hbm.at[c, pl.ds(s * _RT + o, sz)])
            o += sz

    return pl.kernel(
        body,
        out_type=jax.ShapeDtypeStruct((_NC, _NPAD, _D), jnp.float32),
        mesh=_sc_mesh(),
        scratch_types=[
            pltpu.VMEM_SHARED((_NPAD, _D), jnp.float32),
            pltpu.VMEM((8, _D), jnp.float32),
            pltpu.VMEM((_CH,), jnp.int32),
            pltpu.VMEM((_CH,), jnp.int32),
            pltpu.VMEM((_CH,), jnp.float32),
            pltpu.VMEM((_CH, _D), jnp.float32),
            pltpu.SemaphoreType.DMA,
        ],
    )


def _weights_body(comp_ref, basis_ref, out_ref):
    out_ref[0] = jnp.dot(comp_ref[0], basis_ref[0],
                         preferred_element_type=jnp.float32)


def _weights_call(comps, basis2d):
    return pl.pallas_call(
        _weights_body,
        grid=(2,),
        in_specs=[
            pl.BlockSpec((1, _R, _NB), lambda i: (i, 0, 0)),
            pl.BlockSpec((1, _NB, _D * _D), lambda i: (i, 0, 0)),
        ],
        out_specs=pl.BlockSpec((1, _R, _D * _D), lambda i: (i, 0, 0)),
        out_shape=jax.ShapeDtypeStruct((2, _R, _D * _D), jnp.float32),
    )(comps, basis2d)


def _dense1_body(x_ref, w_ref, root_ref, bias_ref, h_ref, xr_ref):
    r = pl.program_id(1)
    xb = x_ref[...]
    h_ref[...] = jnp.dot(xb, w_ref[0], preferred_element_type=jnp.float32)

    @pl.when(r == 0)
    def _():
        xr_ref[...] = (jnp.dot(xb, root_ref[...],
                               preferred_element_type=jnp.float32)
                       + bias_ref[...])


def _dense1_call(x, w3, root, bias2d):
    return pl.pallas_call(
        _dense1_body,
        grid=(_NBLK, _R),
        in_specs=[
            pl.BlockSpec((_BN, _D), lambda nb, r: (nb, 0)),
            pl.BlockSpec((1, _D, _D), lambda nb, r: (r, 0, 0)),
            pl.BlockSpec((_D, _D), lambda nb, r: (0, 0)),
            pl.BlockSpec((1, _D), lambda nb, r: (0, 0)),
        ],
        out_specs=[
            pl.BlockSpec((_BN, _D), lambda nb, r: (r * _NBLK + nb, 0)),
            pl.BlockSpec((_BN, _D), lambda nb, r: (nb, 0)),
        ],
        out_shape=[
            jax.ShapeDtypeStruct((_R * _N, _D), jnp.float32),
            jax.ShapeDtypeStruct((_N, _D), jnp.float32),
        ],
    )(x, w3, root, bias2d)


def _dense2_body(a0_ref, a1_ref, xr1_ref, w_ref, root_ref, bias_ref,
                 h_ref, xr_ref):
    r = pl.program_id(1)
    hb = jnp.maximum(a0_ref[...] + a1_ref[...] + xr1_ref[...], 0.0)
    h_ref[...] = jnp.dot(hb, w_ref[0], preferred_element_type=jnp.float32)

    @pl.when(r == 0)
    def _():
        xr_ref[...] = (jnp.dot(hb, root_ref[...],
                               preferred_element_type=jnp.float32)
                       + bias_ref[...])


def _dense2_call(a0, a1, xr1, w3, root, bias2d):
    return pl.pallas_call(
        _dense2_body,
        grid=(_NBLK, _R),
        in_specs=[
            pl.BlockSpec((_BN, _D), lambda nb, r: (nb, 0)),
            pl.BlockSpec((_BN, _D), lambda nb, r: (nb, 0)),
            pl.BlockSpec((_BN, _D), lambda nb, r: (nb, 0)),
            pl.BlockSpec((1, _D, _D), lambda nb, r: (r, 0, 0)),
            pl.BlockSpec((_D, _D), lambda nb, r: (0, 0)),
            pl.BlockSpec((1, _D), lambda nb, r: (0, 0)),
        ],
        out_specs=[
            pl.BlockSpec((_BN, _D), lambda nb, r: (r * _NBLK + nb, 0)),
            pl.BlockSpec((_BN, _D), lambda nb, r: (nb, 0)),
        ],
        out_shape=[
            jax.ShapeDtypeStruct((_R * _N, _D), jnp.float32),
            jax.ShapeDtypeStruct((_N, _D), jnp.float32),
        ],
    )(a0, a1, xr1, w3, root, bias2d)


def _final_body(a0_ref, a1_ref, xr_ref, o_ref):
    o_ref[...] = a0_ref[...] + a1_ref[...] + xr_ref[...]


def _final_call(a0, a1, xr):
    return pl.pallas_call(
        _final_body,
        grid=(_NBLK,),
        in_specs=[pl.BlockSpec((_BN, _D), lambda nb: (nb, 0))] * 3,
        out_specs=pl.BlockSpec((_BN, _D), lambda nb: (nb, 0)),
        out_shape=jax.ShapeDtypeStruct((_N, _D), jnp.float32),
    )(a0, a1, xr)


def kernel(x, edge_index, edge_type, basis1, comp1, root1, bias1,
           basis2, comp2, root2, bias2):
    E = edge_index.shape[1]
    per_tile = -(-E // (_NW * _CH)) * _CH
    e_pad = per_tile * _NW
    pad = e_pad - E

    src_p = jnp.pad(edge_index[0], (0, pad))
    typ_p = jnp.pad(edge_type, (0, pad))
    dst_p = jnp.pad(edge_index[1], (0, pad), constant_values=_N)

    hist = _make_hist(per_tile)(dst_p, typ_p)
    gidx, w = _make_prep(per_tile)(src_p, typ_p, dst_p,
                                   hist[:_HIST], hist[_HIST:])

    comps = jnp.stack([comp1, comp2])
    basis2d = jnp.stack([basis1.reshape(_NB, _D * _D),
                         basis2.reshape(_NB, _D * _D)])
    w12 = _weights_call(comps, basis2d)
    w1 = w12[0].reshape(_R, _D, _D)
    w2 = w12[1].reshape(_R, _D, _D)

    nchunk = per_tile // _CH
    scatter = _make_scatter(per_tile, (nchunk * 86) // 100)

    h1, xr1 = _dense1_call(x, w1, root1, bias1.reshape(1, _D))
    agg1 = scatter(h1, gidx, w, dst_p)
    h2, xr2 = _dense2_call(agg1[0, :_N], agg1[1, :_N], xr1, w2, root2,
                           bias2.reshape(1, _D))
    agg2 = scatter(h2, gidx, w, dst_p)
    return _final_call(agg2[0, :_N], agg2[1, :_N], xr2)


# speedup vs baseline: 1.1046x; 1.1046x over previous
import functools

import jax
import jax.numpy as jnp
from jax import lax
from jax.experimental import pallas as pl
from jax.experimental.pallas import tpu as pltpu
from jax.experimental.pallas import tpu_sc as plsc

_N = 10000
_R = 16
_D = 128
_NB = 30

_NC = 2
_NS = 16
_NW = _NC * _NS
_CH = 128
_L = 16

_HIST = 160256
_HB = _HIST // _NS
_HW = _HIST // _NW
_NPAD = 10112
_RT = _NPAD // _NS

_BN = 400
_NBLK = _N // _BN


def _sc_mesh():
    return plsc.VectorSubcoreMesh(core_axis_name="c", subcore_axis_name="s")


def _zero16(ref, nwords):
    def z16(j, c2):
        ref[pl.ds(j * _L, _L)] = jnp.zeros((_L,), jnp.float32)
        return c2

    lax.fori_loop(0, nwords // _L, z16, 0)


def _make_hist(per_tile):
    nchunk = per_tile // _CH

    def body(dst_hbm, typ_hbm, out_hbm, hist_sh, histv, dstv, typv, segv,
             onesv, sem):
        c = lax.axis_index("c")
        s = lax.axis_index("s")
        wid = c * _NS + s

        _zero16(histv, _HB)
        pltpu.sync_copy(histv, hist_sh.at[pl.ds(s * _HB, _HB)])
        for j in range(_CH // _L):
            onesv[pl.ds(j * _L, _L)] = jnp.ones((_L,), jnp.float32)
        plsc.subcore_barrier()

        base = wid * per_tile

        def chunk(i, carry):
            off = base + i * _CH
            pltpu.sync_copy(dst_hbm.at[pl.ds(off, _CH)], dstv)
            pltpu.sync_copy(typ_hbm.at[pl.ds(off, _CH)], typv)

            def seg16(j, c2):
                sl = pl.ds(j * _L, _L)
                segv[sl] = dstv[sl] * _R + typv[sl]
                return c2

            lax.fori_loop(0, _CH // _L, seg16, 0)
            pltpu.sync_copy(onesv, hist_sh.at[segv], add=True)
            return carry

        lax.fori_loop(0, nchunk, chunk, 0)
        plsc.subcore_barrier()
        pltpu.sync_copy(hist_sh.at[pl.ds(s * _HB, _HB)], histv)
        pltpu.sync_copy(histv, out_hbm.at[pl.ds(c * _HIST + s * _HB, _HB)])

    return pl.kernel(
        body,
        out_type=jax.ShapeDtypeStruct((_NC * _HIST,), jnp.float32),
        mesh=_sc_mesh(),
        scratch_types=[
            pltpu.VMEM_SHARED((_HIST,), jnp.float32),
            pltpu.VMEM((_HB,), jnp.float32),
            pltpu.VMEM((_CH,), jnp.int32),
            pltpu.VMEM((_CH,), jnp.int32),
            pltpu.VMEM((_CH,), jnp.int32),
            pltpu.VMEM((_CH,), jnp.float32),
            pltpu.SemaphoreType.DMA,
        ],
    )


def _make_prep(per_tile):
    nchunk = per_tile // _CH

    def body(src_hbm, typ_hbm, dst_hbm, h0_hbm, h1_hbm, gidx_hbm, w_hbm,
             srcv, typv, dstv, segv, gidxv, p0v, p1v, wv, sem):
        c = lax.axis_index("c")
        s = lax.axis_index("s")
        wid = c * _NS + s
        base = wid * per_tile

        def chunk(i, carry):
            off = base + i * _CH
            pltpu.sync_copy(src_hbm.at[pl.ds(off, _CH)], srcv)
            pltpu.sync_copy(typ_hbm.at[pl.ds(off, _CH)], typv)
            pltpu.sync_copy(dst_hbm.at[pl.ds(off, _CH)], dstv)

            def idx16(j, c2):
                sl = pl.ds(j * _L, _L)
                gidxv[sl] = typv[sl] * _N + srcv[sl]
                segv[sl] = dstv[sl] * _R + typv[sl]
                return c2

            lax.fori_loop(0, _CH // _L, idx16, 0)
            pltpu.sync_copy(gidxv, gidx_hbm.at[pl.ds(off, _CH)])
            pltpu.async_copy(h0_hbm.at[segv], p0v, sem).wait()
            pltpu.async_copy(h1_hbm.at[segv], p1v, sem).wait()

            def w16(j, c2):
                sl = pl.ds(j * _L, _L)
                wv[sl] = 1.0 / jnp.maximum(p0v[sl] + p1v[sl], 1.0)
                return c2

            lax.fori_loop(0, _CH // _L, w16, 0)
            pltpu.sync_copy(wv, w_hbm.at[pl.ds(off, _CH)])
            return carry

        lax.fori_loop(0, nchunk, chunk, 0)

    return pl.kernel(
        body,
        out_type=(jax.ShapeDtypeStruct((per_tile * _NW,), jnp.int32),
                  jax.ShapeDtypeStruct((per_tile * _NW,), jnp.float32)),
        mesh=_sc_mesh(),
        scratch_types=[
            pltpu.VMEM((_CH,), jnp.int32),
            pltpu.VMEM((_CH,), jnp.int32),
            pltpu.VMEM((_CH,), jnp.int32),
            pltpu.VMEM((_CH,), jnp.int32),
            pltpu.VMEM((_CH,), jnp.int32),
            pltpu.VMEM((_CH,), jnp.float32),
            pltpu.VMEM((_CH,), jnp.float32),
            pltpu.VMEM((_CH,), jnp.float32),
            pltpu.SemaphoreType.DMA,
        ],
    )


_G = 16


def _make_scatter(per_tile, n0):
    nchunk = per_tile // _CH
    n1 = 2 * nchunk - n0

    def body(h_hbm, gidx_hbm, w_hbm, dst_hbm, out_hbm,
             agg_sh, zb, gidxv, dstv, wv_, rows0, sem):
        c = lax.axis_index("c")
        s = lax.axis_index("s")

        for k in range(8):
            for j in range(_D // _L):
                zb[k, pl.ds(j * _L, _L)] = jnp.zeros((_L,), jnp.float32)

        def zrow(i, c2):
            pltpu.sync_copy(zb, agg_sh.at[pl.ds(s * _RT + i * 8, 8)])
            return c2

        lax.fori_loop(0, _RT // 8, zrow, 0)
        plsc.subcore_barrier()

        base = jnp.where(c == 0, s * n0, _NS * n0 + s * n1) * _CH
        nch = jnp.where(c == 0, n0, n1)

        def chunk(i, carry):
            off = base + i * _CH
            pltpu.sync_copy(gidx_hbm.at[pl.ds(off, _CH)], gidxv)
            pltpu.sync_copy(w_hbm.at[pl.ds(off, _CH)], wv_)
            pltpu.sync_copy(dst_hbm.at[pl.ds(off, _CH)], dstv)
            pltpu.async_copy(h_hbm.at[gidxv], rows0, sem).wait()

            def sgrp(gg, c2):
                wv = wv_[pl.ds(gg * _L, _L)]
                for k in range(_L):
                    wsc = wv[k]
                    rr = gg * _L + k
                    for j in range(_D // _L):
                        sl = pl.ds(j * _L, _L)
                        rows0[rr, sl] = rows0[rr, sl] * wsc
                return c2

            lax.fori_loop(0, _CH // _L, sgrp, 0)
            pltpu.sync_copy(rows0, agg_sh.at[dstv], add=True)
            return carry

        lax.fori_loop(0, nch, chunk, 0)
        plsc.subcore_barrier()
        o = 0
        while o < _RT:
            sz = min(_CH, _RT - o)
            pltpu.sync_copy(agg_sh.at[pl.ds(s * _RT + o, sz)],
                            rows0.at[pl.ds(0, sz)])
            pltpu.sync_copy(rows0.at[pl.ds(0, sz)],
                            out_hbm.at[c, pl.ds(s * _RT + o, sz)])
            o += sz

    return pl.kernel(
        body,
        out_type=jax.ShapeDtypeStruct((_NC, _NPAD, _D), jnp.float32),
        mesh=_sc_mesh(),
        scratch_types=[
            pltpu.VMEM_SHARED((_NPAD, _D), jnp.float32),
            pltpu.VMEM((8, _D), jnp.float32),
            pltpu.VMEM((_CH,), jnp.int32),
            pltpu.VMEM((_CH,), jnp.int32),
            pltpu.VMEM((_CH,), jnp.float32),
            pltpu.VMEM((_CH, _D), jnp.float32),
            pltpu.SemaphoreType.DMA,
        ],
    )


def _weights_body(comp_ref, basis_ref, out_ref):
    out_ref[0] = jnp.dot(comp_ref[0], basis_ref[0],
                         preferred_element_type=jnp.float32)


def _weights_call(comps, basis2d):
    return pl.pallas_call(
        _weights_body,
        grid=(2,),
        in_specs=[
            pl.BlockSpec((1, _R, _NB), lambda i: (i, 0, 0)),
            pl.BlockSpec((1, _NB, _D * _D), lambda i: (i, 0, 0)),
        ],
        out_specs=pl.BlockSpec((1, _R, _D * _D), lambda i: (i, 0, 0)),
        out_shape=jax.ShapeDtypeStruct((2, _R, _D * _D), jnp.float32),
    )(comps, basis2d)


def _dense1_body(x_ref, w_ref, root_ref, bias_ref, h_ref, xr_ref):
    r = pl.program_id(1)
    xb = x_ref[...]
    h_ref[...] = jnp.dot(xb, w_ref[0], preferred_element_type=jnp.float32)

    @pl.when(r == 0)
    def _():
        xr_ref[...] = (jnp.dot(xb, root_ref[...],
                               preferred_element_type=jnp.float32)
                       + bias_ref[...])


def _dense1_call(x, w3, root, bias2d):
    return pl.pallas_call(
        _dense1_body,
        grid=(_NBLK, _R),
        in_specs=[
            pl.BlockSpec((_BN, _D), lambda nb, r: (nb, 0)),
            pl.BlockSpec((1, _D, _D), lambda nb, r: (r, 0, 0)),
            pl.BlockSpec((_D, _D), lambda nb, r: (0, 0)),
            pl.BlockSpec((1, _D), lambda nb, r: (0, 0)),
        ],
        out_specs=[
            pl.BlockSpec((_BN, _D), lambda nb, r: (r * _NBLK + nb, 0)),
            pl.BlockSpec((_BN, _D), lambda nb, r: (nb, 0)),
        ],
        out_shape=[
            jax.ShapeDtypeStruct((_R * _N, _D), jnp.float32),
            jax.ShapeDtypeStruct((_N, _D), jnp.float32),
        ],
    )(x, w3, root, bias2d)


def _dense2_body(a0_ref, a1_ref, xr1_ref, w_ref, root_ref, bias_ref,
                 h_ref, xr_ref):
    r = pl.program_id(1)
    hb = jnp.maximum(a0_ref[...] + a1_ref[...] + xr1_ref[...], 0.0)
    h_ref[...] = jnp.dot(hb, w_ref[0], preferred_element_type=jnp.float32)

    @pl.when(r == 0)
    def _():
        xr_ref[...] = (jnp.dot(hb, root_ref[...],
                               preferred_element_type=jnp.float32)
                       + bias_ref[...])


def _dense2_call(a0, a1, xr1, w3, root, bias2d):
    return pl.pallas_call(
        _dense2_body,
        grid=(_NBLK, _R),
        in_specs=[
            pl.BlockSpec((_BN, _D), lambda nb, r: (nb, 0)),
            pl.BlockSpec((_BN, _D), lambda nb, r: (nb, 0)),
            pl.BlockSpec((_BN, _D), lambda nb, r: (nb, 0)),
            pl.BlockSpec((1, _D, _D), lambda nb, r: (r, 0, 0)),
            pl.BlockSpec((_D, _D), lambda nb, r: (0, 0)),
            pl.BlockSpec((1, _D), lambda nb, r: (0, 0)),
        ],
        out_specs=[
            pl.BlockSpec((_BN, _D), lambda nb, r: (r * _NBLK + nb, 0)),
            pl.BlockSpec((_BN, _D), lambda nb, r: (nb, 0)),
        ],
        out_shape=[
            jax.ShapeDtypeStruct((_R * _N, _D), jnp.float32),
            jax.ShapeDtypeStruct((_N, _D), jnp.float32),
        ],
    )(a0, a1, xr1, w3, root, bias2d)


def _final_body(a0_ref, a1_ref, xr_ref, o_ref):
    o_ref[...] = a0_ref[...] + a1_ref[...] + xr_ref[...]


def _final_call(a0, a1, xr):
    return pl.pallas_call(
        _final_body,
        grid=(_NBLK,),
        in_specs=[pl.BlockSpec((_BN, _D), lambda nb: (nb, 0))] * 3,
        out_specs=pl.BlockSpec((_BN, _D), lambda nb: (nb, 0)),
        out_shape=jax.ShapeDtypeStruct((_N, _D), jnp.float32),
    )(a0, a1, xr)


def kernel(x, edge_index, edge_type, basis1, comp1, root1, bias1,
           basis2, comp2, root2, bias2):
    E = edge_index.shape[1]
    per_tile = -(-E // (_NW * _CH)) * _CH
    e_pad = per_tile * _NW
    pad = e_pad - E

    src_p = jnp.pad(edge_index[0], (0, pad))
    typ_p = jnp.pad(edge_type, (0, pad))
    dst_p = jnp.pad(edge_index[1], (0, pad), constant_values=_N)

    hist = _make_hist(per_tile)(dst_p, typ_p)
    gidx, w = _make_prep(per_tile)(src_p, typ_p, dst_p,
                                   hist[:_HIST], hist[_HIST:])

    comps = jnp.stack([comp1, comp2])
    basis2d = jnp.stack([basis1.reshape(_NB, _D * _D),
                         basis2.reshape(_NB, _D * _D)])
    w12 = _weights_call(comps, basis2d)
    w1 = w12[0].reshape(_R, _D, _D)
    w2 = w12[1].reshape(_R, _D, _D)

    nchunk = per_tile // _CH
    scatter = _make_scatter(per_tile, (nchunk * 114) // 100)

    h1, xr1 = _dense1_call(x, w1, root1, bias1.reshape(1, _D))
    agg1 = scatter(h1, gidx, w, dst_p)
    h2, xr2 = _dense2_call(agg1[0, :_N], agg1[1, :_N], xr1, w2, root2,
                           bias2.reshape(1, _D))
    agg2 = scatter(h2, gidx, w, dst_p)
    return _final_call(agg2[0, :_N], agg2[1, :_N], xr2)


# speedup vs baseline: 1.1187x; 1.0127x over previous
import functools

import jax
import jax.numpy as jnp
from jax import lax
from jax.experimental import pallas as pl
from jax.experimental.pallas import tpu as pltpu
from jax.experimental.pallas import tpu_sc as plsc

_N = 10000
_R = 16
_D = 128
_NB = 30

_NC = 2
_NS = 16
_NW = _NC * _NS
_CH = 128
_L = 16

_HIST = 160256
_HB = _HIST // _NS
_HW = _HIST // _NW
_NPAD = 10112
_RT = _NPAD // _NS

_BN = 400
_NBLK = _N // _BN


def _sc_mesh():
    return plsc.VectorSubcoreMesh(core_axis_name="c", subcore_axis_name="s")


def _zero16(ref, nwords):
    def z16(j, c2):
        ref[pl.ds(j * _L, _L)] = jnp.zeros((_L,), jnp.float32)
        return c2

    lax.fori_loop(0, nwords // _L, z16, 0)


def _make_hist(per_tile):
    nchunk = per_tile // _CH

    def body(dst_hbm, typ_hbm, out_hbm, hist_sh, histv, dstv, typv, segv,
             onesv, sem):
        c = lax.axis_index("c")
        s = lax.axis_index("s")
        wid = c * _NS + s

        _zero16(histv, _HB)
        pltpu.sync_copy(histv, hist_sh.at[pl.ds(s * _HB, _HB)])
        for j in range(_CH // _L):
            onesv[pl.ds(j * _L, _L)] = jnp.ones((_L,), jnp.float32)
        plsc.subcore_barrier()

        base = wid * per_tile

        def chunk(i, carry):
            off = base + i * _CH
            pltpu.sync_copy(dst_hbm.at[pl.ds(off, _CH)], dstv)
            pltpu.sync_copy(typ_hbm.at[pl.ds(off, _CH)], typv)

            def seg16(j, c2):
                sl = pl.ds(j * _L, _L)
                segv[sl] = dstv[sl] * _R + typv[sl]
                return c2

            lax.fori_loop(0, _CH // _L, seg16, 0)
            pltpu.sync_copy(onesv, hist_sh.at[segv], add=True)
            return carry

        lax.fori_loop(0, nchunk, chunk, 0)
        plsc.subcore_barrier()
        pltpu.sync_copy(hist_sh.at[pl.ds(s * _HB, _HB)], histv)
        pltpu.sync_copy(histv, out_hbm.at[pl.ds(c * _HIST + s * _HB, _HB)])

    return pl.kernel(
        body,
        out_type=jax.ShapeDtypeStruct((_NC * _HIST,), jnp.float32),
        mesh=_sc_mesh(),
        scratch_types=[
            pltpu.VMEM_SHARED((_HIST,), jnp.float32),
            pltpu.VMEM((_HB,), jnp.float32),
            pltpu.VMEM((_CH,), jnp.int32),
            pltpu.VMEM((_CH,), jnp.int32),
            pltpu.VMEM((_CH,), jnp.int32),
            pltpu.VMEM((_CH,), jnp.float32),
            pltpu.SemaphoreType.DMA,
        ],
    )


def _make_prep(per_tile):
    nchunk = per_tile // _CH

    def body(src_hbm, typ_hbm, dst_hbm, h0_hbm, h1_hbm, gidx_hbm, w_hbm,
             srcv, typv, dstv, segv, gidxv, p0v, p1v, wv, sem):
        c = lax.axis_index("c")
        s = lax.axis_index("s")
        wid = c * _NS + s
        base = wid * per_tile

        def chunk(i, carry):
            off = base + i * _CH
            pltpu.sync_copy(src_hbm.at[pl.ds(off, _CH)], srcv)
            pltpu.sync_copy(typ_hbm.at[pl.ds(off, _CH)], typv)
            pltpu.sync_copy(dst_hbm.at[pl.ds(off, _CH)], dstv)

            def idx16(j, c2):
                sl = pl.ds(j * _L, _L)
                gidxv[sl] = typv[sl] * _N + srcv[sl]
                segv[sl] = dstv[sl] * _R + typv[sl]
                return c2

            lax.fori_loop(0, _CH // _L, idx16, 0)
            pltpu.sync_copy(gidxv, gidx_hbm.at[pl.ds(off, _CH)])
            pltpu.async_copy(h0_hbm.at[segv], p0v, sem).wait()
            pltpu.async_copy(h1_hbm.at[segv], p1v, sem).wait()

            def w16(j, c2):
                sl = pl.ds(j * _L, _L)
                wv[sl] = 1.0 / jnp.maximum(p0v[sl] + p1v[sl], 1.0)
                return c2

            lax.fori_loop(0, _CH // _L, w16, 0)
            pltpu.sync_copy(wv, w_hbm.at[pl.ds(off, _CH)])
            return carry

        lax.fori_loop(0, nchunk, chunk, 0)

    return pl.kernel(
        body,
        out_type=(jax.ShapeDtypeStruct((per_tile * _NW,), jnp.int32),
                  jax.ShapeDtypeStruct((per_tile * _NW,), jnp.float32)),
        mesh=_sc_mesh(),
        scratch_types=[
            pltpu.VMEM((_CH,), jnp.int32),
            pltpu.VMEM((_CH,), jnp.int32),
            pltpu.VMEM((_CH,), jnp.int32),
            pltpu.VMEM((_CH,), jnp.int32),
            pltpu.VMEM((_CH,), jnp.int32),
            pltpu.VMEM((_CH,), jnp.float32),
            pltpu.VMEM((_CH,), jnp.float32),
            pltpu.VMEM((_CH,), jnp.float32),
            pltpu.SemaphoreType.DMA,
        ],
    )


_G = 16


def _make_scatter(per_tile, n0):
    nchunk = per_tile // _CH
    n1 = 2 * nchunk - n0

    def body(h_hbm, gidx_hbm, w_hbm, dst_hbm, out_hbm,
             agg_sh, zb, gidxv, dstv, wv_, rows0, sem):
        c = lax.axis_index("c")
        s = lax.axis_index("s")

        for k in range(8):
            for j in range(_D // _L):
                zb[k, pl.ds(j * _L, _L)] = jnp.zeros((_L,), jnp.float32)

        def zrow(i, c2):
            pltpu.sync_copy(zb, agg_sh.at[pl.ds(s * _RT + i * 8, 8)])
            return c2

        lax.fori_loop(0, _RT // 8, zrow, 0)
        plsc.subcore_barrier()

        base = jnp.where(c == 0, s * n0, _NS * n0 + s * n1) * _CH
        nch = jnp.where(c == 0, n0, n1)

        def chunk(i, carry):
            off = base + i * _CH
            pltpu.sync_copy(gidx_hbm.at[pl.ds(off, _CH)], gidxv)
            pltpu.sync_copy(w_hbm.at[pl.ds(off, _CH)], wv_)
            pltpu.sync_copy(dst_hbm.at[pl.ds(off, _CH)], dstv)
            pltpu.async_copy(h_hbm.at[gidxv], rows0, sem).wait()

            def sgrp(gg, c2):
                wv = wv_[pl.ds(gg * _L, _L)]
                for k in range(_L):
                    wsc = wv[k]
                    rr = gg * _L + k
                    for j in range(_D // _L):
                        sl = pl.ds(j * _L, _L)
                        rows0[rr, sl] = rows0[rr, sl] * wsc
                return c2

            lax.fori_loop(0, _CH // _L, sgrp, 0)
            pltpu.sync_copy(rows0, agg_sh.at[dstv], add=True)
            return carry

        lax.fori_loop(0, nch, chunk, 0)
        plsc.subcore_barrier()
        o = 0
        while o < _RT:
            sz = min(_CH, _RT - o)
            pltpu.sync_copy(agg_sh.at[pl.ds(s * _RT + o, sz)],
                            rows0.at[pl.ds(0, sz)])
            pltpu.sync_copy(rows0.at[pl.ds(0, sz)],
                            out_hbm.at[c, pl.ds(s * _RT + o, sz)])
            o += sz

    return pl.kernel(
        body,
        out_type=jax.ShapeDtypeStruct((_NC, _NPAD, _D), jnp.float32),
        mesh=_sc_mesh(),
        scratch_types=[
            pltpu.VMEM_SHARED((_NPAD, _D), jnp.float32),
            pltpu.VMEM((8, _D), jnp.float32),
            pltpu.VMEM((_CH,), jnp.int32),
            pltpu.VMEM((_CH,), jnp.int32),
            pltpu.VMEM((_CH,), jnp.float32),
            pltpu.VMEM((_CH, _D), jnp.float32),
            pltpu.SemaphoreType.DMA,
        ],
    )


def _weights_body(comp_ref, basis_ref, out_ref):
    out_ref[0] = jnp.dot(comp_ref[0], basis_ref[0],
                         preferred_element_type=jnp.float32)


def _weights_call(comps, basis2d):
    return pl.pallas_call(
        _weights_body,
        grid=(2,),
        in_specs=[
            pl.BlockSpec((1, _R, _NB), lambda i: (i, 0, 0)),
            pl.BlockSpec((1, _NB, _D * _D), lambda i: (i, 0, 0)),
        ],
        out_specs=pl.BlockSpec((1, _R, _D * _D), lambda i: (i, 0, 0)),
        out_shape=jax.ShapeDtypeStruct((2, _R, _D * _D), jnp.float32),
    )(comps, basis2d)


def _dense1_body(x_ref, w_ref, root_ref, bias_ref, h_ref, xr_ref):
    r = pl.program_id(1)
    xb = x_ref[...]
    h_ref[...] = jnp.dot(xb, w_ref[0], preferred_element_type=jnp.float32)

    @pl.when(r == 0)
    def _():
        xr_ref[...] = (jnp.dot(xb, root_ref[...],
                               preferred_element_type=jnp.float32)
                       + bias_ref[...])


def _dense1_call(x, w3, root, bias2d):
    return pl.pallas_call(
        _dense1_body,
        grid=(_NBLK, _R),
        in_specs=[
            pl.BlockSpec((_BN, _D), lambda nb, r: (nb, 0)),
            pl.BlockSpec((1, _D, _D), lambda nb, r: (r, 0, 0)),
            pl.BlockSpec((_D, _D), lambda nb, r: (0, 0)),
            pl.BlockSpec((1, _D), lambda nb, r: (0, 0)),
        ],
        out_specs=[
            pl.BlockSpec((_BN, _D), lambda nb, r: (r * _NBLK + nb, 0)),
            pl.BlockSpec((_BN, _D), lambda nb, r: (nb, 0)),
        ],
        out_shape=[
            jax.ShapeDtypeStruct((_R * _N, _D), jnp.float32),
            jax.ShapeDtypeStruct((_N, _D), jnp.float32),
        ],
    )(x, w3, root, bias2d)


def _dense2_body(a0_ref, a1_ref, xr1_ref, w_ref, root_ref, bias_ref,
                 h_ref, xr_ref):
    r = pl.program_id(1)
    hb = jnp.maximum(a0_ref[...] + a1_ref[...] + xr1_ref[...], 0.0)
    h_ref[...] = jnp.dot(hb, w_ref[0], preferred_element_type=jnp.float32)

    @pl.when(r == 0)
    def _():
        xr_ref[...] = (jnp.dot(hb, root_ref[...],
                               preferred_element_type=jnp.float32)
                       + bias_ref[...])


def _dense2_call(a0, a1, xr1, w3, root, bias2d):
    return pl.pallas_call(
        _dense2_body,
        grid=(_NBLK, _R),
        in_specs=[
            pl.BlockSpec((_BN, _D), lambda nb, r: (nb, 0)),
            pl.BlockSpec((_BN, _D), lambda nb, r: (nb, 0)),
            pl.BlockSpec((_BN, _D), lambda nb, r: (nb, 0)),
            pl.BlockSpec((1, _D, _D), lambda nb, r: (r, 0, 0)),
            pl.BlockSpec((_D, _D), lambda nb, r: (0, 0)),
            pl.BlockSpec((1, _D), lambda nb, r: (0, 0)),
        ],
        out_specs=[
            pl.BlockSpec((_BN, _D), lambda nb, r: (r * _NBLK + nb, 0)),
            pl.BlockSpec((_BN, _D), lambda nb, r: (nb, 0)),
        ],
        out_shape=[
            jax.ShapeDtypeStruct((_R * _N, _D), jnp.float32),
            jax.ShapeDtypeStruct((_N, _D), jnp.float32),
        ],
    )(a0, a1, xr1, w3, root, bias2d)


def _final_body(a0_ref, a1_ref, xr_ref, o_ref):
    o_ref[...] = a0_ref[...] + a1_ref[...] + xr_ref[...]


def _final_call(a0, a1, xr):
    return pl.pallas_call(
        _final_body,
        grid=(_NBLK,),
        in_specs=[pl.BlockSpec((_BN, _D), lambda nb: (nb, 0))] * 3,
        out_specs=pl.BlockSpec((_BN, _D), lambda nb: (nb, 0)),
        out_shape=jax.ShapeDtypeStruct((_N, _D), jnp.float32),
    )(a0, a1, xr)


def kernel(x, edge_index, edge_type, basis1, comp1, root1, bias1,
           basis2, comp2, root2, bias2):
    E = edge_index.shape[1]
    per_tile = -(-E // (_NW * _CH)) * _CH
    e_pad = per_tile * _NW
    pad = e_pad - E

    src_p = jnp.pad(edge_index[0], (0, pad))
    typ_p = jnp.pad(edge_type, (0, pad))
    dst_p = jnp.pad(edge_index[1], (0, pad), constant_values=_N)

    hist = _make_hist(per_tile)(dst_p, typ_p)
    gidx, w = _make_prep(per_tile)(src_p, typ_p, dst_p,
                                   hist[:_HIST], hist[_HIST:])

    comps = jnp.stack([comp1, comp2])
    basis2d = jnp.stack([basis1.reshape(_NB, _D * _D),
                         basis2.reshape(_NB, _D * _D)])
    w12 = _weights_call(comps, basis2d)
    w1 = w12[0].reshape(_R, _D, _D)
    w2 = w12[1].reshape(_R, _D, _D)

    nchunk = per_tile // _CH
    scatter = _make_scatter(per_tile, (nchunk * 119) // 100)

    h1, xr1 = _dense1_call(x, w1, root1, bias1.reshape(1, _D))
    agg1 = scatter(h1, gidx, w, dst_p)
    h2, xr2 = _dense2_call(agg1[0, :_N], agg1[1, :_N], xr1, w2, root2,
                           bias2.reshape(1, _D))
    agg2 = scatter(h2, gidx, w, dst_p)
    return _final_call(agg2[0, :_N], agg2[1, :_N], xr2)
